# Initial kernel scaffold; baseline (speedup 1.0000x reference)
import jax
import jax.numpy as jnp
from jax.experimental import pallas as pl


def kernel(nids, nfeats):
    raise NotImplementedError("write your pallas kernel here")


# speedup vs baseline: 1.6915x; 1.6915x over previous
import functools

import jax
import jax.numpy as jnp
from jax import lax
from jax.experimental import pallas as pl
from jax.experimental.pallas import tpu as pltpu
from jax.experimental.pallas import tpu_sc as plsc

VOCAB = 1000000
EMBED_DIM = 64
BATCH = 16384
HIST = 50

try:
    _info = plsc.get_sparse_core_info()
    _NC, _NS = _info.num_cores, _info.num_subcores
except Exception:
    _NC, _NS = 2, 16

_NW = _NC * _NS
_B = BATCH * HIST
_CHUNK = 128
_PER_W = _B // _NW
_NCHUNK = _PER_W // _CHUNK

assert _PER_W * _NW == _B and _NCHUNK * _CHUNK == _PER_W


def _make_gather():
    mesh = plsc.VectorSubcoreMesh(core_axis_name="c", subcore_axis_name="s")

    @functools.partial(
        pl.kernel,
        mesh=mesh,
        out_type=jax.ShapeDtypeStruct((_B, EMBED_DIM), jnp.float32),
        scratch_types=[
            pltpu.VMEM((_NCHUNK, _CHUNK), jnp.int32),
            pltpu.VMEM((_CHUNK, EMBED_DIM), jnp.float32),
            pltpu.SemaphoreType.DMA,
        ],
        compiler_params=pltpu.CompilerParams(use_tc_tiling_on_sc=False),
    )
    def gather_kernel(idx_hbm, table_hbm, out_hbm, idx_v, rows_v, gsem):
        wid = lax.axis_index("s") * _NC + lax.axis_index("c")
        base = wid * _PER_W
        pltpu.sync_copy(idx_hbm.at[wid], idx_v)

        @pl.loop(0, _NCHUNK)
        def _chunk(c):
            pltpu.async_copy(table_hbm.at[idx_v.at[c]], rows_v, gsem).wait()
            pltpu.sync_copy(rows_v, out_hbm.at[pl.ds(base + c * _CHUNK, _CHUNK)])

    return gather_kernel


_gather = _make_gather()


def kernel(nids, nfeats):
    idx = nids.reshape(_NW, _NCHUNK, _CHUNK).astype(jnp.int32)
    out = _gather(idx, nfeats)
    return out.reshape(BATCH, HIST, EMBED_DIM)


# speedup vs baseline: 1.8752x; 1.1086x over previous
import functools

import jax
import jax.numpy as jnp
from jax import lax
from jax.experimental import pallas as pl
from jax.experimental.pallas import tpu as pltpu
from jax.experimental.pallas import tpu_sc as plsc

VOCAB = 1000000
EMBED_DIM = 64
BATCH = 16384
HIST = 50

try:
    _info = plsc.get_sparse_core_info()
    _NC, _NS = _info.num_cores, _info.num_subcores
except Exception:
    _NC, _NS = 2, 16

_NW = _NC * _NS
_B = BATCH * HIST
_CHUNK = 128
_PER_W = _B // _NW
_NCHUNK = _PER_W // _CHUNK

assert _PER_W * _NW == _B and _NCHUNK * _CHUNK == _PER_W


_NBUF = 8
_GLAG = 4
_NGRP = _NCHUNK // _NBUF

assert _NGRP * _NBUF == _NCHUNK


def _make_gather():
    mesh = plsc.VectorSubcoreMesh(core_axis_name="c", subcore_axis_name="s")

    @functools.partial(
        pl.kernel,
        mesh=mesh,
        out_type=jax.ShapeDtypeStruct((_B, EMBED_DIM), jnp.float32),
        scratch_types=(
            [pltpu.VMEM((_NCHUNK, _CHUNK), jnp.int32)]
            + [pltpu.VMEM((_CHUNK, EMBED_DIM), jnp.float32)] * _NBUF
            + [pltpu.SemaphoreType.DMA] * (2 * _NBUF)
        ),
        compiler_params=pltpu.CompilerParams(use_tc_tiling_on_sc=False),
    )
    def gather_kernel(idx_hbm, table_hbm, out_hbm, idx_v, *scratch):
        rows = scratch[:_NBUF]
        gsem = scratch[_NBUF:2 * _NBUF]
        osem = scratch[2 * _NBUF:]

        wid = lax.axis_index("s") * _NC + lax.axis_index("c")
        base = wid * _PER_W
        pltpu.sync_copy(idx_hbm.at[wid], idx_v)

        def gather_issue(c, b):
            pltpu.async_copy(table_hbm.at[idx_v.at[c]], rows[b], gsem[b])

        def gather_wait(b):
            pltpu.make_async_copy(
                table_hbm.at[idx_v.at[0]], rows[b], gsem[b]).wait()

        def write_issue(c, b):
            pltpu.async_copy(
                rows[b], out_hbm.at[pl.ds(base + c * _CHUNK, _CHUNK)], osem[b])

        def write_wait(b):
            pltpu.make_async_copy(
                rows[b], out_hbm.at[pl.ds(base, _CHUNK)], osem[b]).wait()

        for s in range(_NBUF):
            gather_issue(s, s)
            if s >= _GLAG:
                gather_wait(s - _GLAG)
                write_issue(s - _GLAG, s - _GLAG)

        @pl.loop(1, _NGRP)
        def _group(t):
            s0 = t * _NBUF
            for j in range(_NBUF):
                write_wait(j)
                gather_issue(s0 + j, j)
                b = (j + _GLAG) % _NBUF
                gather_wait(b)
                write_issue(s0 + j - _GLAG, b)

        for j in range(_GLAG):
            b = (j + _GLAG) % _NBUF
            gather_wait(b)
            write_issue(_NCHUNK - _GLAG + j, b)
        for j in range(_NBUF):
            write_wait(j)

    return gather_kernel


_gather = _make_gather()


def kernel(nids, nfeats):
    idx = nids.reshape(_NW, _NCHUNK, _CHUNK).astype(jnp.int32)
    out = _gather(idx, nfeats)
    return out.reshape(BATCH, HIST, EMBED_DIM)
